# baseline (speedup 1.0000x reference)
import functools

import jax
import jax.numpy as jnp
from jax import lax
from jax.experimental import pallas as pl
from jax.experimental.pallas import tpu as pltpu
from jax.experimental.pallas import tpu_sc as plsc

HIDDEN = 64
CH = 128
NC = 2
NS = 16
NW = NC * NS
NBUF = 2


def _gather(table2, idx3d, seq, batch):
    rows_per_w = idx3d.shape[1]
    ngroups = rows_per_w // NBUF
    bblocks = batch // CH

    mesh = plsc.VectorSubcoreMesh(
        core_axis_name="c", subcore_axis_name="s", num_cores=NC, num_subcores=NS
    )

    @functools.partial(
        pl.kernel,
        out_type=jax.ShapeDtypeStruct((seq, HIDDEN, batch), jnp.float32),
        mesh=mesh,
        scratch_types=[
            pltpu.VMEM((rows_per_w, CH), jnp.int32),
            pltpu.VMEM((rows_per_w, CH), jnp.int32),
            pltpu.VMEM((NBUF, CH, 2 * HIDDEN), jnp.float32),
            pltpu.VMEM((NBUF, HIDDEN, CH), jnp.float32),
            pltpu.SemaphoreType.DMA((NBUF,)),
            pltpu.SemaphoreType.DMA((NBUF,)),
        ],
        compiler_params=pltpu.CompilerParams(
            use_tc_tiling_on_sc=True, needs_layout_passes=False
        ),
    )
    def k(table_hbm, idx_hbm, out_hbm, idx_v, pair_v, rows_v, rowst_v, gsem, wsem):
        wid = lax.axis_index("s") * NC + lax.axis_index("c")
        row0 = wid * rows_per_w
        pltpu.sync_copy(idx_hbm.at[wid], idx_v)

        lane = lax.iota(jnp.int32, 16)

        def mk_pairs(i, carry):
            for g in range(CH // 16):
                pair_v[i, pl.ds(g * 16, 16)] = (
                    idx_v[i, pl.ds(g * 16, 16)] >> 1
                )
            return carry

        lax.fori_loop(0, rows_per_w, mk_pairs, 0)

        def fire_gather(i, b):
            pltpu.async_copy(table_hbm.at[pair_v.at[i]], rows_v.at[b], gsem.at[b])

        def wait_gather(b):
            pltpu.make_async_copy(
                table_hbm.at[pl.ds(0, CH)], rows_v.at[b], gsem.at[b]
            ).wait()

        def fire_write(i, b):
            c = row0 + i
            s = c // bblocks
            bb = c % bblocks
            pltpu.async_copy(
                rowst_v.at[b], out_hbm.at[s, :, pl.ds(bb * CH, CH)], wsem.at[b]
            )

        def wait_write(b):
            pltpu.make_async_copy(
                rowst_v.at[b], out_hbm.at[0, :, pl.ds(0, CH)], wsem.at[b]
            ).wait()

        def select_transpose(i, b):
            for g in range(CH // 16):
                rvec = lane + g * 16
                half = (idx_v[i, pl.ds(g * 16, 16)] & 1) * HIDDEN

                @plsc.parallel_loop(0, HIDDEN, 1, unroll=8)
                def _(h):
                    vals = plsc.load_gather(rows_v.at[b], [rvec, half + h])
                    rowst_v[b, h, pl.ds(g * 16, 16)] = vals

        for b in range(NBUF):
            fire_gather(b, b)

        def group(g, carry):
            for b in range(NBUF):
                i = g * NBUF + b
                wait_gather(b)

                @pl.when(g > 0)
                def _():
                    wait_write(b)

                select_transpose(i, b)
                fire_write(i, b)

                @pl.when(g + 1 < ngroups)
                def _():
                    fire_gather(i + NBUF, b)

            return carry

        lax.fori_loop(0, ngroups, group, 0)
        for b in range(NBUF):
            wait_write(b)

    return k(table2, idx3d)


def kernel(token_ids, key, embed_table):
    b, s = token_ids.shape
    v, h = embed_table.shape
    tab_t = jnp.transpose(embed_table)
    tab3 = jnp.transpose(jnp.reshape(tab_t, (h, v // 2, 2)), (1, 2, 0))
    tab2 = jnp.reshape(tab3, (v // 2, 2 * h))
    tok_t = jnp.transpose(token_ids.astype(jnp.int32))
    idx3d = jnp.reshape(tok_t, (NW, s * b // (NW * CH), CH))
    out_phys = _gather(tab2, idx3d, s, b)
    return jnp.transpose(out_phys, (2, 0, 1))


# speedup vs baseline: 1.2025x; 1.2025x over previous
import functools

import jax
import jax.numpy as jnp
from jax import lax
from jax.experimental import pallas as pl
from jax.experimental.pallas import tpu as pltpu
from jax.experimental.pallas import tpu_sc as plsc

HIDDEN = 64
CH = 128
NC = 2
NS = 16
NW = NC * NS
NBUF = 4


def _gather(table2, idx3d, seq, batch):
    rows_per_w = idx3d.shape[1]
    ngroups = rows_per_w // NBUF
    bblocks = batch // CH

    mesh = plsc.VectorSubcoreMesh(
        core_axis_name="c", subcore_axis_name="s", num_cores=NC, num_subcores=NS
    )

    @functools.partial(
        pl.kernel,
        out_type=jax.ShapeDtypeStruct((seq, HIDDEN, batch), jnp.float32),
        mesh=mesh,
        scratch_types=[
            pltpu.VMEM((rows_per_w, CH), jnp.int32),
            pltpu.VMEM((rows_per_w, CH), jnp.int32),
            pltpu.VMEM((NBUF, CH, 2 * HIDDEN), jnp.float32),
            pltpu.VMEM((NBUF, HIDDEN, CH), jnp.float32),
            pltpu.SemaphoreType.DMA((NBUF,)),
            pltpu.SemaphoreType.DMA((NBUF,)),
        ],
        compiler_params=pltpu.CompilerParams(
            use_tc_tiling_on_sc=True, needs_layout_passes=False
        ),
    )
    def k(table_hbm, idx_hbm, out_hbm, idx_v, pair_v, rows_v, rowst_v, gsem, wsem):
        wid = lax.axis_index("s") * NC + lax.axis_index("c")
        row0 = wid * rows_per_w
        pltpu.sync_copy(idx_hbm.at[wid], idx_v)

        lane = lax.iota(jnp.int32, 16)

        def mk_pairs(i, carry):
            for g in range(CH // 16):
                pair_v[i, pl.ds(g * 16, 16)] = (
                    idx_v[i, pl.ds(g * 16, 16)] >> 1
                )
            return carry

        lax.fori_loop(0, rows_per_w, mk_pairs, 0)

        def fire_gather(i, b):
            pltpu.async_copy(table_hbm.at[pair_v.at[i]], rows_v.at[b], gsem.at[b])

        def wait_gather(b):
            pltpu.make_async_copy(
                table_hbm.at[pl.ds(0, CH)], rows_v.at[b], gsem.at[b]
            ).wait()

        def fire_write(i, b):
            c = row0 + i
            s = c // bblocks
            bb = c % bblocks
            pltpu.async_copy(
                rowst_v.at[b], out_hbm.at[s, :, pl.ds(bb * CH, CH)], wsem.at[b]
            )

        def wait_write(b):
            pltpu.make_async_copy(
                rowst_v.at[b], out_hbm.at[0, :, pl.ds(0, CH)], wsem.at[b]
            ).wait()

        def select_transpose(i, b):
            for g in range(CH // 16):
                rvec = lane + g * 16
                half = (idx_v[i, pl.ds(g * 16, 16)] & 1) * HIDDEN

                @plsc.parallel_loop(0, HIDDEN, 1, unroll=16)
                def _(h):
                    vals = plsc.load_gather(rows_v.at[b], [rvec, half + h])
                    rowst_v[b, h, pl.ds(g * 16, 16)] = vals

        for b in range(NBUF):
            fire_gather(b, b)

        def group(g, carry):
            for b in range(NBUF):
                i = g * NBUF + b
                wait_gather(b)

                @pl.when(g > 0)
                def _():
                    wait_write(b)

                select_transpose(i, b)
                fire_write(i, b)

                @pl.when(g + 1 < ngroups)
                def _():
                    fire_gather(i + NBUF, b)

            return carry

        lax.fori_loop(0, ngroups, group, 0)
        for b in range(NBUF):
            wait_write(b)

    return k(table2, idx3d)


def kernel(token_ids, key, embed_table):
    b, s = token_ids.shape
    v, h = embed_table.shape
    tab2 = jnp.reshape(embed_table, (v // 2, 2 * h))
    tok_t = jnp.transpose(token_ids.astype(jnp.int32))
    idx3d = jnp.reshape(tok_t, (NW, s * b // (NW * CH), CH))
    out_phys = _gather(tab2, idx3d, s, b)
    return jnp.transpose(out_phys, (2, 0, 1))
